# Initial kernel scaffold; baseline (speedup 1.0000x reference)
import jax
import jax.numpy as jnp
from jax.experimental import pallas as pl


def kernel(edge_index, edge_weight, node_feat, W1, b1, W2, b2):
    raise NotImplementedError("write your pallas kernel here")


# speedup vs baseline: 4.5217x; 4.5217x over previous
import functools

import jax
import jax.numpy as jnp
from jax import lax
from jax.experimental import pallas as pl
from jax.experimental.pallas import tpu as pltpu
from jax.experimental.pallas import tpu_sc as plsc

_N = 10000
_E = 320000
_D = 128
_NC = 2
_NS = 16
_NW = _NC * _NS
_EPW = _E // _NW
_B = 80
_NB = _EPW // _B
_NCHUNK = _N // _B
_CPT = -(-_NCHUNK // _NS)
_LANES = _D // 16


def _sc_mesh():
    return plsc.VectorSubcoreMesh(
        core_axis_name="c", subcore_axis_name="s",
        num_cores=_NC, num_subcores=_NS)


@functools.partial(
    pl.kernel,
    out_type=jax.ShapeDtypeStruct((_NC * _N, _D), jnp.float32),
    mesh=_sc_mesh(),
    scratch_types=[
        pltpu.VMEM_SHARED((_N, _D), jnp.float32),
        pltpu.VMEM((_B,), jnp.int32),
        pltpu.VMEM((_B,), jnp.int32),
        pltpu.VMEM((_B,), jnp.float32),
        pltpu.VMEM((_B, _D), jnp.float32),
        pltpu.SemaphoreType.DMA,
    ],
)
def _aggregate(src_hbm, dst_hbm, w_hbm, feat_hbm, out_hbm,
               agg_sh, src_v, dst_v, w_v, rows_v, sem):
    c = lax.axis_index("c")
    s = lax.axis_index("s")

    def _zero_row(i, carry):
        for k in range(_LANES):
            rows_v[i, pl.ds(k * 16, 16)] = jnp.zeros((16,), jnp.float32)
        return carry
    lax.fori_loop(0, _B, _zero_row, 0)
    for j in range(_CPT):
        cid = j * _NS + s
        if (j + 1) * _NS > _NCHUNK:
            @pl.when(cid < _NCHUNK)
            def _():
                pltpu.sync_copy(rows_v, agg_sh.at[pl.ds(cid * _B, _B)])
        else:
            pltpu.sync_copy(rows_v, agg_sh.at[pl.ds(cid * _B, _B)])
    plsc.subcore_barrier()

    ebase = (c * _NS + s) * _EPW

    def _batch(j, carry):
        b0 = ebase + j * _B
        pltpu.sync_copy(src_hbm.at[pl.ds(b0, _B)], src_v)
        pltpu.sync_copy(dst_hbm.at[pl.ds(b0, _B)], dst_v)
        pltpu.sync_copy(w_hbm.at[pl.ds(b0, _B)], w_v)
        pltpu.async_copy(feat_hbm.at[src_v], rows_v, sem).wait()

        def _scale(g, inner):
            wvec = w_v[pl.ds(g * 16, 16)]
            for b in range(16):
                wb = lax.gather(
                    wvec, jnp.full((16, 1), b, jnp.int32),
                    lax.GatherDimensionNumbers(
                        offset_dims=(), collapsed_slice_dims=(0,),
                        start_index_map=(0,)),
                    (1,), mode=lax.GatherScatterMode.PROMISE_IN_BOUNDS)
                r = g * 16 + b
                for k in range(_LANES):
                    rows_v[r, pl.ds(k * 16, 16)] = wb * rows_v[r, pl.ds(k * 16, 16)]
            return inner
        lax.fori_loop(0, _B // 16, _scale, 0)
        pltpu.sync_copy(rows_v, agg_sh.at[dst_v], add=True)
        return carry
    lax.fori_loop(0, _NB, _batch, 0)
    plsc.subcore_barrier()

    for j in range(_CPT):
        cid = j * _NS + s

        def _drain(cid=cid):
            pltpu.sync_copy(agg_sh.at[pl.ds(cid * _B, _B)], rows_v)
            pltpu.sync_copy(rows_v, out_hbm.at[pl.ds(c * _N + cid * _B, _B)])
        if (j + 1) * _NS > _NCHUNK:
            pl.when(cid < _NCHUNK)(_drain)
        else:
            _drain()


def _dense_body(p_ref, nf_ref, w1t_ref, w2t_ref, b1_ref, b2_ref, o_ref):
    a = p_ref[0] + p_ref[1]
    nf = nf_ref[...]
    y1 = jnp.dot(a, w1t_ref[...], preferred_element_type=jnp.float32) + b1_ref[...]
    y2 = jnp.dot(a * nf, w2t_ref[...], preferred_element_type=jnp.float32) + b2_ref[...]
    o_ref[...] = (jnp.where(y1 >= 0, y1, 0.2 * y1)
                  + jnp.where(y2 >= 0, y2, 0.2 * y2))


_ROWS_BLK = 1000


def _dense(p, node_feat, w1t, w2t, b1, b2):
    grid = (_N // _ROWS_BLK,)
    return pl.pallas_call(
        _dense_body,
        grid=grid,
        in_specs=[
            pl.BlockSpec((_NC, _ROWS_BLK, _D), lambda i: (0, i, 0)),
            pl.BlockSpec((_ROWS_BLK, _D), lambda i: (i, 0)),
            pl.BlockSpec((_D, _D), lambda i: (0, 0)),
            pl.BlockSpec((_D, _D), lambda i: (0, 0)),
            pl.BlockSpec((1, _D), lambda i: (0, 0)),
            pl.BlockSpec((1, _D), lambda i: (0, 0)),
        ],
        out_specs=pl.BlockSpec((_ROWS_BLK, _D), lambda i: (i, 0)),
        out_shape=jax.ShapeDtypeStruct((_N, _D), jnp.float32),
    )(p, node_feat, w1t, w2t, b1, b2)


def kernel(edge_index, edge_weight, node_feat, W1, b1, W2, b2):
    dst = edge_index[0].astype(jnp.int32)
    src = edge_index[1].astype(jnp.int32)
    w = edge_weight.astype(jnp.float32)
    partials = _aggregate(src, dst, w, node_feat)
    p = partials.reshape(_NC, _N, _D)
    return _dense(p, node_feat, W1.T, W2.T,
                  b1.reshape(1, _D), b2.reshape(1, _D))
